# baseline (speedup 1.0000x reference)
import functools

import jax
import jax.numpy as jnp
from jax.experimental import pallas as pl
from jax.experimental.pallas import tpu as pltpu
from jax.experimental.pallas import tpu_sc as plsc

_ACT = 768
_TOPK = 64
_L1_COEFF = 0.0008
_INF_BITS = 0x7F800000

_BM = 256
_BN = 1536
_BK = 1536

_LADDER = (0, 0x3D800000, 0x3F000000, 0x3F400000, 0x3F800000, 0x41000000)
_D = len(_LADDER)

_G = 16
_NW = 32
_NB = 2048
_CH = 128


def _encode_kernel(x_ref, w_ref, b_ref, sel_ref, acts_ref, gmax_ref,
                   cnt_ref):
    i = pl.program_id(0)
    j = pl.program_id(1)
    xc = x_ref[...] - b_ref[...]
    a = jnp.maximum(
        jnp.dot(xc, w_ref[...], preferred_element_type=jnp.float32), 0.0
    )
    acts_ref[...] = a

    m = a
    for sh in (8, 4, 2, 1):
        m = jnp.maximum(m, pltpu.roll(m, sh, axis=1))
    gmax_ref[...] = jnp.dot(m, sel_ref[...],
                            preferred_element_type=jnp.float32)[None]

    @pl.when(jnp.logical_and(i == 0, j == 0))
    def _init():
        for t in range(_D):
            cnt_ref[t] = 0

    bits = jax.lax.bitcast_convert_type(a, jnp.int32)
    for t in range(_D):
        cnt_ref[t] += jnp.sum(bits > _LADDER[t], dtype=jnp.int32)


def _sc_select_body(params_hbm, gmax_hbm, acts_hbm, hist_hbm,
                    params_v, gmaxs_v, idx_v, rows_v, hist_v, outb_v, sem):
    gpw = gmax_hbm.shape[0] // _NW
    cid = jax.lax.axis_index("c")
    sid = jax.lax.axis_index("s")
    wid = sid * 2 + cid
    lane = jax.lax.iota(jnp.int32, 16)

    pltpu.sync_copy(params_hbm, params_v)
    pv = params_v[...]
    zero = jnp.int32(0)
    lo = jnp.sum(jnp.where(lane == 0, pv, zero))
    s = jnp.sum(jnp.where(lane == 1, pv, zero))

    pltpu.sync_copy(gmax_hbm.at[pl.ds(wid * gpw, gpw)], gmaxs_v)

    def zh(i, c):
        hist_v[pl.ds(i * 16, 16)] = jnp.zeros((16,), jnp.int32)
        return c

    jax.lax.fori_loop(0, (16 * _NB) // 16, zh, 0, unroll=8)

    pad = jnp.full((16,), wid * gpw, jnp.int32)

    def zi(i, c):
        idx_v[pl.ds(i * 16, 16)] = pad
        return c

    jax.lax.fori_loop(0, gpw // 16, zi, 0, unroll=8)

    def scan(i, off_v):
        g = plsc.bitcast(gmaxs_v[pl.ds(i * 16, 16)], jnp.int32)
        m = g > lo
        inc = jnp.where(m, jnp.int32(1), jnp.int32(0))
        pos = off_v + plsc.cumsum(inc) - 1
        ids = (wid * gpw + i * 16) + lane
        plsc.store_scatter(idx_v, [pos], ids, mask=m)
        return off_v + plsc.all_reduce_population_count(m)

    off_v = jax.lax.fori_loop(
        0, gpw // 16, scan, jnp.zeros((16,), jnp.int32), unroll=4
    )
    n = jnp.sum(jnp.where(lane == 0, off_v, zero))

    ntrip = (n + _CH - 1) // _CH
    ones = jnp.ones((16,), jnp.int32)
    lanenb = lane * _NB

    def chunk(c2, carry):
        pltpu.async_copy(
            acts_hbm.at[idx_v.at[pl.ds(c2 * _CH, _CH)]], rows_v, sem
        ).wait()
        nvalid = n - c2 * _CH

        def hrow(j, c3):
            v = plsc.bitcast(rows_v[j, :], jnp.int32)
            m = jnp.logical_and(v > lo, j < nvalid)
            rel = v - (lo + 1)
            b = jnp.minimum(jax.lax.shift_right_logical(rel, s), _NB - 1)
            plsc.addupdate_scatter(hist_v, [b + lanenb], ones, mask=m)
            return c3

        jax.lax.fori_loop(0, _CH, hrow, 0, unroll=4)
        return carry

    jax.lax.fori_loop(0, ntrip, chunk, 0)

    def merge(q, c):
        acc = jnp.zeros((16,), jnp.int32)
        for l in range(16):
            acc = acc + hist_v[pl.ds(l * _NB + q * 16, 16)]
        outb_v[pl.ds(q * 16, 16)] = acc
        return c

    jax.lax.fori_loop(0, _NB // 16, merge, 0)
    pltpu.sync_copy(outb_v, hist_hbm.at[wid])


def _decode_kernel(vk_ref, acts_ref, w_ref, b_ref, x_ref,
                   topk_ref, sae_ref, stat_ref, acc_ref, nbt, nkt):
    i = pl.program_id(0)
    kt = pl.program_id(1)
    a = acts_ref[...]
    bits = jax.lax.bitcast_convert_type(a, jnp.int32)
    m = jnp.where(bits >= vk_ref[0], a, 0.0)
    topk_ref[...] = m

    @pl.when(kt == 0)
    def _zero_acc():
        acc_ref[...] = jnp.zeros_like(acc_ref)

    acc_ref[...] += jnp.dot(m, w_ref[...], preferred_element_type=jnp.float32)

    @pl.when(jnp.logical_and(i == 0, kt == 0))
    def _init_stats():
        stat_ref[0] = 0.0
        stat_ref[1] = 0.0
        stat_ref[2] = jnp.inf
        stat_ref[3] = 0.0

    stat_ref[0] += jnp.sum(m)
    stat_ref[1] += jnp.sum(m > 0, dtype=jnp.float32)
    stat_ref[2] = jnp.minimum(
        stat_ref[2], jnp.min(jnp.where(m > 0, m, jnp.inf))
    )

    @pl.when(kt == nkt - 1)
    def _finish_row():
        sae = acc_ref[...] + b_ref[...]
        sae_ref[...] = sae
        d = sae - x_ref[...]
        stat_ref[3] += jnp.sum(d * d)


def _bracket_update(thrs, hi, counts, k, dead):
    ge = counts >= k
    idx = jnp.arange(_D, dtype=jnp.int32)
    jstar = jnp.max(jnp.where(ge, idx, -1))
    dead = jnp.logical_or(dead, jstar < 0)
    js = jnp.maximum(jstar, 0)
    new_lo = thrs[js]
    new_hi = jnp.where(js < _D - 1, thrs[jnp.minimum(js + 1, _D - 1)], hi)
    return new_lo, new_hi, dead


def _sc_update(lo, hi, s, hist, k, dead):
    tot = jnp.sum(hist, axis=0)
    suf = jnp.cumsum(tot[::-1])[::-1]
    ge = suf >= k
    idx = jnp.arange(_NB, dtype=jnp.int32)
    b = jnp.max(jnp.where(ge, idx, -1))
    dead = jnp.logical_or(dead, b < 0)
    bs = jnp.maximum(b, 0)
    w = hi - lo
    new_lo = lo + (bs << s)
    new_hi = lo + jnp.minimum((bs + 1) << s, w)
    return new_lo, new_hi, dead


def kernel(x, W_enc, W_dec, b_dec):
    batch = x.shape[0]
    dict_size = W_enc.shape[1]
    k = _TOPK * batch
    b2 = b_dec.reshape(1, _ACT)

    nbt = batch // _BM
    ndt = dict_size // _BN
    ng = _BN // _G
    cidx = jnp.arange(ng)
    sel = jnp.zeros((_BN, ng), jnp.float32).at[cidx * _G + _G - 1, cidx].set(1.0)

    acts, gmax, cnt0 = pl.pallas_call(
        _encode_kernel,
        grid=(nbt, ndt),
        in_specs=[
            pl.BlockSpec((_BM, _ACT), lambda i, j: (i, 0)),
            pl.BlockSpec((_ACT, _BN), lambda i, j: (0, j)),
            pl.BlockSpec((1, _ACT), lambda i, j: (0, 0)),
            pl.BlockSpec((_BN, ng), lambda i, j: (0, 0)),
        ],
        out_specs=[
            pl.BlockSpec((_BM, _BN), lambda i, j: (i, j)),
            pl.BlockSpec((1, _BM, ng), lambda i, j: (j, i, 0)),
            pl.BlockSpec(memory_space=pltpu.SMEM),
        ],
        out_shape=[
            jax.ShapeDtypeStruct((batch, dict_size), jnp.float32),
            jax.ShapeDtypeStruct((ndt, batch, ng), jnp.float32),
            jax.ShapeDtypeStruct((_D,), jnp.int32),
        ],
    )(x, W_enc, b2, sel)

    thrs0 = jnp.array(_LADDER, dtype=jnp.int32)
    lo, hi, dead = _bracket_update(
        thrs0, jnp.int32(_INF_BITS), cnt0, k, jnp.bool_(False)
    )

    ngrp = (batch * dict_size) // _G
    acts2d = acts.reshape(ngrp, _G)
    gmaxf = gmax.transpose(1, 0, 2).reshape(ngrp)

    sc_round = pl.kernel(
        _sc_select_body,
        out_type=jax.ShapeDtypeStruct((_NW, _NB), jnp.int32),
        mesh=plsc.VectorSubcoreMesh(core_axis_name="c", subcore_axis_name="s"),
        compiler_params=pltpu.CompilerParams(
            needs_layout_passes=False, use_tc_tiling_on_sc=False),
        scratch_types=[
            pltpu.VMEM((16,), jnp.int32),
            pltpu.VMEM((ngrp // _NW,), jnp.float32),
            pltpu.VMEM((ngrp // _NW,), jnp.int32),
            pltpu.VMEM((_CH, _G), jnp.float32),
            pltpu.VMEM((16 * _NB,), jnp.int32),
            pltpu.VMEM((_NB,), jnp.int32),
            pltpu.SemaphoreType.DMA,
        ],
    )

    for _ in range(3):
        w = hi - lo
        s = jnp.maximum(0, (32 - jax.lax.clz(w - 1)) - 11)
        params = jnp.zeros((16,), jnp.int32).at[0].set(lo).at[1].set(s)
        hist = sc_round(params, gmaxf, acts2d)
        lo, hi, dead = _sc_update(lo, hi, s, hist, k, dead)

    vk_bits = jnp.where(dead, jnp.int32(0), hi)

    nkt = dict_size // _BK
    topk, sae, stats = pl.pallas_call(
        functools.partial(_decode_kernel, nbt=nbt, nkt=nkt),
        grid=(nbt, nkt),
        in_specs=[
            pl.BlockSpec(memory_space=pltpu.SMEM),
            pl.BlockSpec((_BM, _BK), lambda i, t: (i, t)),
            pl.BlockSpec((_BK, _ACT), lambda i, t: (t, 0)),
            pl.BlockSpec((1, _ACT), lambda i, t: (0, 0)),
            pl.BlockSpec((_BM, _ACT), lambda i, t: (i, 0)),
        ],
        out_specs=[
            pl.BlockSpec((_BM, _BK), lambda i, t: (i, t)),
            pl.BlockSpec((_BM, _ACT), lambda i, t: (i, 0)),
            pl.BlockSpec(memory_space=pltpu.SMEM),
        ],
        out_shape=[
            jax.ShapeDtypeStruct((batch, dict_size), jnp.float32),
            jax.ShapeDtypeStruct((batch, _ACT), jnp.float32),
            jax.ShapeDtypeStruct((8,), jnp.float32),
        ],
        scratch_shapes=[pltpu.VMEM((_BM, _ACT), jnp.float32)],
    )(vk_bits.reshape(1), acts, W_dec, b2, x)

    l1_norm = stats[0] / batch
    l0_norm = stats[1] / batch
    minpos = stats[2]
    l2_loss = stats[3] / (batch * _ACT)
    l1_loss = jnp.float32(_L1_COEFF) * l1_norm
    loss = l2_loss + l1_loss
    threshold = jnp.where(jnp.isfinite(minpos), minpos, jnp.float32(0.0))

    return (sae, topk, loss, l2_loss, l1_loss, l0_norm, l1_norm, threshold)


# speedup vs baseline: 1.0131x; 1.0131x over previous
import functools

import jax
import jax.numpy as jnp
from jax.experimental import pallas as pl
from jax.experimental.pallas import tpu as pltpu
from jax.experimental.pallas import tpu_sc as plsc

_ACT = 768
_TOPK = 64
_L1_COEFF = 0.0008
_INF_BITS = 0x7F800000

_BM = 256
_BN = 1536
_BK = 1536

_LADDER = (0, 0x3D800000, 0x3F000000, 0x3F400000, 0x3F800000, 0x41000000)
_D = len(_LADDER)

_G = 16
_NW = 32
_NB = 1024
_CH = 128


def _encode_kernel(x_ref, w_ref, b_ref, sel_ref, acts_ref, gmax_ref,
                   cnt_ref):
    i = pl.program_id(0)
    j = pl.program_id(1)
    xc = x_ref[...] - b_ref[...]
    a = jnp.maximum(
        jnp.dot(xc, w_ref[...], preferred_element_type=jnp.float32), 0.0
    )
    acts_ref[...] = a

    m = a
    for sh in (8, 4, 2, 1):
        m = jnp.maximum(m, pltpu.roll(m, sh, axis=1))
    gmax_ref[...] = jnp.dot(m, sel_ref[...],
                            preferred_element_type=jnp.float32)[None]

    @pl.when(jnp.logical_and(i == 0, j == 0))
    def _init():
        for t in range(_D):
            cnt_ref[t] = 0

    bits = jax.lax.bitcast_convert_type(a, jnp.int32)
    for t in range(_D):
        cnt_ref[t] += jnp.sum(bits > _LADDER[t], dtype=jnp.int32)


def _sc_select_body(params_hbm, gmax_hbm, acts_hbm, hist_hbm,
                    params_v, gmaxs_v, idx_v, rows_v, hist_v, outb_v, sem):
    gpw = gmax_hbm.shape[0] // _NW
    cid = jax.lax.axis_index("c")
    sid = jax.lax.axis_index("s")
    wid = sid * 2 + cid
    lane = jax.lax.iota(jnp.int32, 16)

    pltpu.sync_copy(params_hbm, params_v)
    pv = params_v[...]
    zero = jnp.int32(0)
    lo = jnp.sum(jnp.where(lane == 0, pv, zero))
    s = jnp.sum(jnp.where(lane == 1, pv, zero))

    pltpu.sync_copy(gmax_hbm.at[pl.ds(wid * gpw, gpw)], gmaxs_v)

    def zh(i, c):
        hist_v[pl.ds(i * 16, 16)] = jnp.zeros((16,), jnp.int32)
        return c

    jax.lax.fori_loop(0, (16 * _NB) // 16, zh, 0, unroll=8)

    def scan(i, off_v):
        g = plsc.bitcast(gmaxs_v[pl.ds(i * 16, 16)], jnp.int32)
        m = g > lo
        inc = jnp.where(m, jnp.int32(1), jnp.int32(0))
        pos = off_v + plsc.cumsum(inc) - 1
        ids = (wid * gpw + i * 16) + lane
        plsc.store_scatter(idx_v, [pos], ids, mask=m)
        return off_v + plsc.all_reduce_population_count(m)

    off_v = jax.lax.fori_loop(
        0, gpw // 16, scan, jnp.zeros((16,), jnp.int32), unroll=4
    )
    n = jnp.sum(jnp.where(lane == 0, off_v, zero))

    padv = jnp.full((16,), wid * gpw, jnp.int32)
    for pj in range(_CH // 16):
        posp = n + pj * 16 + lane
        plsc.store_scatter(idx_v, [posp], padv, mask=posp < gpw)

    ntrip = (n + _CH - 1) // _CH
    ones = jnp.ones((16,), jnp.int32)
    lanenb = lane * _NB

    def chunk(c2, carry):
        pltpu.async_copy(
            acts_hbm.at[idx_v.at[pl.ds(c2 * _CH, _CH)]], rows_v, sem
        ).wait()
        nvalid = n - c2 * _CH

        def hrow(j, c3):
            v = plsc.bitcast(rows_v[j, :], jnp.int32)
            m = jnp.logical_and(v > lo, j < nvalid)
            rel = v - (lo + 1)
            b = jnp.minimum(jax.lax.shift_right_logical(rel, s), _NB - 1)
            plsc.addupdate_scatter(hist_v, [b + lanenb], ones, mask=m)
            return c3

        jax.lax.fori_loop(0, _CH, hrow, 0, unroll=4)
        return carry

    jax.lax.fori_loop(0, ntrip, chunk, 0)

    def merge(q, c):
        acc = jnp.zeros((16,), jnp.int32)
        for l in range(16):
            acc = acc + hist_v[pl.ds(l * _NB + q * 16, 16)]
        outb_v[pl.ds(q * 16, 16)] = acc
        return c

    jax.lax.fori_loop(0, _NB // 16, merge, 0)
    pltpu.sync_copy(outb_v, hist_hbm.at[wid])


def _decode_kernel(vk_ref, acts_ref, w_ref, b_ref, x_ref,
                   topk_ref, sae_ref, stat_ref, acc_ref, nbt, nkt):
    i = pl.program_id(0)
    kt = pl.program_id(1)
    a = acts_ref[...]
    bits = jax.lax.bitcast_convert_type(a, jnp.int32)
    m = jnp.where(bits >= vk_ref[0], a, 0.0)
    topk_ref[...] = m

    @pl.when(kt == 0)
    def _zero_acc():
        acc_ref[...] = jnp.zeros_like(acc_ref)

    acc_ref[...] += jnp.dot(m, w_ref[...], preferred_element_type=jnp.float32)

    @pl.when(jnp.logical_and(i == 0, kt == 0))
    def _init_stats():
        stat_ref[0] = 0.0
        stat_ref[1] = 0.0
        stat_ref[2] = jnp.inf
        stat_ref[3] = 0.0

    stat_ref[0] += jnp.sum(m)
    stat_ref[1] += jnp.sum(m > 0, dtype=jnp.float32)
    stat_ref[2] = jnp.minimum(
        stat_ref[2], jnp.min(jnp.where(m > 0, m, jnp.inf))
    )

    @pl.when(kt == nkt - 1)
    def _finish_row():
        sae = acc_ref[...] + b_ref[...]
        sae_ref[...] = sae
        d = sae - x_ref[...]
        stat_ref[3] += jnp.sum(d * d)


def _bracket_update(thrs, hi, counts, k, dead):
    ge = counts >= k
    idx = jnp.arange(_D, dtype=jnp.int32)
    jstar = jnp.max(jnp.where(ge, idx, -1))
    dead = jnp.logical_or(dead, jstar < 0)
    js = jnp.maximum(jstar, 0)
    new_lo = thrs[js]
    new_hi = jnp.where(js < _D - 1, thrs[jnp.minimum(js + 1, _D - 1)], hi)
    return new_lo, new_hi, dead


def _sc_update(lo, hi, s, hist, k, dead):
    tot = jnp.sum(hist, axis=0)
    suf = jnp.cumsum(tot[::-1])[::-1]
    ge = suf >= k
    idx = jnp.arange(_NB, dtype=jnp.int32)
    b = jnp.max(jnp.where(ge, idx, -1))
    dead = jnp.logical_or(dead, b < 0)
    bs = jnp.maximum(b, 0)
    w = hi - lo
    new_lo = lo + (bs << s)
    new_hi = lo + jnp.minimum((bs + 1) << s, w)
    return new_lo, new_hi, dead


def kernel(x, W_enc, W_dec, b_dec):
    batch = x.shape[0]
    dict_size = W_enc.shape[1]
    k = _TOPK * batch
    b2 = b_dec.reshape(1, _ACT)

    nbt = batch // _BM
    ndt = dict_size // _BN
    ng = _BN // _G
    cidx = jnp.arange(ng)
    sel = jnp.zeros((_BN, ng), jnp.float32).at[cidx * _G + _G - 1, cidx].set(1.0)

    acts, gmax, cnt0 = pl.pallas_call(
        _encode_kernel,
        grid=(nbt, ndt),
        in_specs=[
            pl.BlockSpec((_BM, _ACT), lambda i, j: (i, 0)),
            pl.BlockSpec((_ACT, _BN), lambda i, j: (0, j)),
            pl.BlockSpec((1, _ACT), lambda i, j: (0, 0)),
            pl.BlockSpec((_BN, ng), lambda i, j: (0, 0)),
        ],
        out_specs=[
            pl.BlockSpec((_BM, _BN), lambda i, j: (i, j)),
            pl.BlockSpec((1, _BM, ng), lambda i, j: (j, i, 0)),
            pl.BlockSpec(memory_space=pltpu.SMEM),
        ],
        out_shape=[
            jax.ShapeDtypeStruct((batch, dict_size), jnp.float32),
            jax.ShapeDtypeStruct((ndt, batch, ng), jnp.float32),
            jax.ShapeDtypeStruct((_D,), jnp.int32),
        ],
    )(x, W_enc, b2, sel)

    thrs0 = jnp.array(_LADDER, dtype=jnp.int32)
    lo, hi, dead = _bracket_update(
        thrs0, jnp.int32(_INF_BITS), cnt0, k, jnp.bool_(False)
    )

    ngrp = (batch * dict_size) // _G
    acts2d = acts.reshape(ngrp, _G)
    gmaxf = gmax.transpose(1, 0, 2).reshape(ngrp)

    sc_round = pl.kernel(
        _sc_select_body,
        out_type=jax.ShapeDtypeStruct((_NW, _NB), jnp.int32),
        mesh=plsc.VectorSubcoreMesh(core_axis_name="c", subcore_axis_name="s"),
        compiler_params=pltpu.CompilerParams(
            needs_layout_passes=False, use_tc_tiling_on_sc=False),
        scratch_types=[
            pltpu.VMEM((16,), jnp.int32),
            pltpu.VMEM((ngrp // _NW,), jnp.float32),
            pltpu.VMEM((ngrp // _NW,), jnp.int32),
            pltpu.VMEM((_CH, _G), jnp.float32),
            pltpu.VMEM((16 * _NB,), jnp.int32),
            pltpu.VMEM((_NB,), jnp.int32),
            pltpu.SemaphoreType.DMA,
        ],
    )

    for _ in range(3):
        w = hi - lo
        s = jnp.maximum(0, (32 - jax.lax.clz(w - 1)) - 10)
        params = jnp.zeros((16,), jnp.int32).at[0].set(lo).at[1].set(s)
        hist = sc_round(params, gmaxf, acts2d)
        lo, hi, dead = _sc_update(lo, hi, s, hist, k, dead)

    vk_bits = jnp.where(dead, jnp.int32(0), hi)

    nkt = dict_size // _BK
    topk, sae, stats = pl.pallas_call(
        functools.partial(_decode_kernel, nbt=nbt, nkt=nkt),
        grid=(nbt, nkt),
        in_specs=[
            pl.BlockSpec(memory_space=pltpu.SMEM),
            pl.BlockSpec((_BM, _BK), lambda i, t: (i, t)),
            pl.BlockSpec((_BK, _ACT), lambda i, t: (t, 0)),
            pl.BlockSpec((1, _ACT), lambda i, t: (0, 0)),
            pl.BlockSpec((_BM, _ACT), lambda i, t: (i, 0)),
        ],
        out_specs=[
            pl.BlockSpec((_BM, _BK), lambda i, t: (i, t)),
            pl.BlockSpec((_BM, _ACT), lambda i, t: (i, 0)),
            pl.BlockSpec(memory_space=pltpu.SMEM),
        ],
        out_shape=[
            jax.ShapeDtypeStruct((batch, dict_size), jnp.float32),
            jax.ShapeDtypeStruct((batch, _ACT), jnp.float32),
            jax.ShapeDtypeStruct((8,), jnp.float32),
        ],
        scratch_shapes=[pltpu.VMEM((_BM, _ACT), jnp.float32)],
    )(vk_bits.reshape(1), acts, W_dec, b2, x)

    l1_norm = stats[0] / batch
    l0_norm = stats[1] / batch
    minpos = stats[2]
    l2_loss = stats[3] / (batch * _ACT)
    l1_loss = jnp.float32(_L1_COEFF) * l1_norm
    loss = l2_loss + l1_loss
    threshold = jnp.where(jnp.isfinite(minpos), minpos, jnp.float32(0.0))

    return (sae, topk, loss, l2_loss, l1_loss, l0_norm, l1_norm, threshold)


# speedup vs baseline: 1.1264x; 1.1119x over previous
import functools

import jax
import jax.numpy as jnp
from jax.experimental import pallas as pl
from jax.experimental.pallas import tpu as pltpu
from jax.experimental.pallas import tpu_sc as plsc

_ACT = 768
_TOPK = 64
_L1_COEFF = 0.0008
_INF_BITS = 0x7F800000

_BM = 256
_BN = 1536
_BK = 1536

_LADDER = (0, 0x3F000000, 0x3F628F5C, 0x3F6CCCCD, 0x3FC00000, 0x41000000)
_D = len(_LADDER)

_G = 16
_NW = 32
_NB = 1024
_CH = 128


def _encode_kernel(x_ref, w_ref, b_ref, sel_ref, acts_ref, gmax_ref,
                   cnt_ref):
    i = pl.program_id(0)
    j = pl.program_id(1)
    xc = x_ref[...] - b_ref[...]
    a = jnp.maximum(
        jnp.dot(xc, w_ref[...], preferred_element_type=jnp.float32), 0.0
    )
    acts_ref[...] = a

    m = a
    for sh in (8, 4, 2, 1):
        m = jnp.maximum(m, pltpu.roll(m, sh, axis=1))
    gmax_ref[...] = jnp.dot(m, sel_ref[...],
                            preferred_element_type=jnp.float32)[None]

    @pl.when(jnp.logical_and(i == 0, j == 0))
    def _init():
        for t in range(_D):
            cnt_ref[t] = 0

    bits = jax.lax.bitcast_convert_type(a, jnp.int32)
    for t in range(_D):
        cnt_ref[t] += jnp.sum(bits > _LADDER[t], dtype=jnp.int32)


def _sc_select_body(params_hbm, gmax_hbm, acts_hbm, hist_hbm,
                    params_v, gmaxs_v, idx_v, rows_v, hist_v, outb_v, sem):
    gpw = gmax_hbm.shape[0] // _NW
    cid = jax.lax.axis_index("c")
    sid = jax.lax.axis_index("s")
    wid = sid * 2 + cid
    lane = jax.lax.iota(jnp.int32, 16)

    pltpu.sync_copy(params_hbm, params_v)
    pv = params_v[...]
    zero = jnp.int32(0)
    lo = jnp.sum(jnp.where(lane == 0, pv, zero))
    s = jnp.sum(jnp.where(lane == 1, pv, zero))

    pltpu.sync_copy(gmax_hbm.at[pl.ds(wid * gpw, gpw)], gmaxs_v)

    def zh(i, c):
        hist_v[pl.ds(i * 16, 16)] = jnp.zeros((16,), jnp.int32)
        return c

    jax.lax.fori_loop(0, (16 * _NB) // 16, zh, 0, unroll=8)

    def scan(i, off_v):
        g = plsc.bitcast(gmaxs_v[pl.ds(i * 16, 16)], jnp.int32)
        m = g > lo
        inc = jnp.where(m, jnp.int32(1), jnp.int32(0))
        pos = off_v + plsc.cumsum(inc) - 1
        ids = (wid * gpw + i * 16) + lane
        plsc.store_scatter(idx_v, [pos], ids, mask=m)
        return off_v + plsc.all_reduce_population_count(m)

    off_v = jax.lax.fori_loop(
        0, gpw // 16, scan, jnp.zeros((16,), jnp.int32), unroll=4
    )
    n = jnp.sum(jnp.where(lane == 0, off_v, zero))

    padv = jnp.full((16,), wid * gpw, jnp.int32)
    for pj in range(_CH // 16):
        posp = n + pj * 16 + lane
        plsc.store_scatter(idx_v, [posp], padv, mask=posp < gpw)

    ntrip = (n + _CH - 1) // _CH
    ones = jnp.ones((16,), jnp.int32)
    lanenb = lane * _NB

    def chunk(c2, carry):
        pltpu.async_copy(
            acts_hbm.at[idx_v.at[pl.ds(c2 * _CH, _CH)]], rows_v, sem
        ).wait()
        nvalid = n - c2 * _CH

        def hrow(j, c3):
            v = plsc.bitcast(rows_v[j, :], jnp.int32)
            m = jnp.logical_and(v > lo, j < nvalid)
            rel = v - (lo + 1)
            b = jnp.minimum(jax.lax.shift_right_logical(rel, s), _NB - 1)
            plsc.addupdate_scatter(hist_v, [b + lanenb], ones, mask=m)
            return c3

        jax.lax.fori_loop(0, _CH, hrow, 0, unroll=4)
        return carry

    jax.lax.fori_loop(0, ntrip, chunk, 0)

    def merge(q, c):
        acc = jnp.zeros((16,), jnp.int32)
        for l in range(16):
            acc = acc + hist_v[pl.ds(l * _NB + q * 16, 16)]
        outb_v[pl.ds(q * 16, 16)] = acc
        return c

    jax.lax.fori_loop(0, _NB // 16, merge, 0)
    pltpu.sync_copy(outb_v, hist_hbm.at[wid])


def _decode_kernel(vk_ref, acts_ref, w_ref, b_ref, x_ref,
                   topk_ref, sae_ref, stat_ref, acc_ref, nbt, nkt):
    i = pl.program_id(0)
    kt = pl.program_id(1)
    a = acts_ref[...]
    bits = jax.lax.bitcast_convert_type(a, jnp.int32)
    m = jnp.where(bits >= vk_ref[0], a, 0.0)
    topk_ref[...] = m

    @pl.when(kt == 0)
    def _zero_acc():
        acc_ref[...] = jnp.zeros_like(acc_ref)

    acc_ref[...] += jnp.dot(m, w_ref[...], preferred_element_type=jnp.float32)

    @pl.when(jnp.logical_and(i == 0, kt == 0))
    def _init_stats():
        stat_ref[0] = 0.0
        stat_ref[1] = 0.0
        stat_ref[2] = jnp.inf
        stat_ref[3] = 0.0

    stat_ref[0] += jnp.sum(m)
    stat_ref[1] += jnp.sum(m > 0, dtype=jnp.float32)
    stat_ref[2] = jnp.minimum(
        stat_ref[2], jnp.min(jnp.where(m > 0, m, jnp.inf))
    )

    @pl.when(kt == nkt - 1)
    def _finish_row():
        sae = acc_ref[...] + b_ref[...]
        sae_ref[...] = sae
        d = sae - x_ref[...]
        stat_ref[3] += jnp.sum(d * d)


def _bracket_update(thrs, hi, counts, k, dead):
    ge = counts >= k
    idx = jnp.arange(_D, dtype=jnp.int32)
    jstar = jnp.max(jnp.where(ge, idx, -1))
    dead = jnp.logical_or(dead, jstar < 0)
    js = jnp.maximum(jstar, 0)
    new_lo = thrs[js]
    new_hi = jnp.where(js < _D - 1, thrs[jnp.minimum(js + 1, _D - 1)], hi)
    return new_lo, new_hi, dead


def _sc_update(lo, hi, s, hist, k, dead):
    tot = jnp.sum(hist, axis=0)
    suf = jnp.cumsum(tot[::-1])[::-1]
    ge = suf >= k
    idx = jnp.arange(_NB, dtype=jnp.int32)
    b = jnp.max(jnp.where(ge, idx, -1))
    dead = jnp.logical_or(dead, b < 0)
    bs = jnp.maximum(b, 0)
    w = hi - lo
    new_lo = lo + (bs << s)
    new_hi = lo + jnp.minimum((bs + 1) << s, w)
    return new_lo, new_hi, dead


def kernel(x, W_enc, W_dec, b_dec):
    batch = x.shape[0]
    dict_size = W_enc.shape[1]
    k = _TOPK * batch
    b2 = b_dec.reshape(1, _ACT)

    nbt = batch // _BM
    ndt = dict_size // _BN
    ng = _BN // _G
    cidx = jnp.arange(ng)
    sel = jnp.zeros((_BN, ng), jnp.float32).at[cidx * _G + _G - 1, cidx].set(1.0)

    acts, gmax, cnt0 = pl.pallas_call(
        _encode_kernel,
        grid=(nbt, ndt),
        in_specs=[
            pl.BlockSpec((_BM, _ACT), lambda i, j: (i, 0)),
            pl.BlockSpec((_ACT, _BN), lambda i, j: (0, j)),
            pl.BlockSpec((1, _ACT), lambda i, j: (0, 0)),
            pl.BlockSpec((_BN, ng), lambda i, j: (0, 0)),
        ],
        out_specs=[
            pl.BlockSpec((_BM, _BN), lambda i, j: (i, j)),
            pl.BlockSpec((1, _BM, ng), lambda i, j: (j, i, 0)),
            pl.BlockSpec(memory_space=pltpu.SMEM),
        ],
        out_shape=[
            jax.ShapeDtypeStruct((batch, dict_size), jnp.float32),
            jax.ShapeDtypeStruct((ndt, batch, ng), jnp.float32),
            jax.ShapeDtypeStruct((_D,), jnp.int32),
        ],
    )(x, W_enc, b2, sel)

    thrs0 = jnp.array(_LADDER, dtype=jnp.int32)
    lo, hi, dead = _bracket_update(
        thrs0, jnp.int32(_INF_BITS), cnt0, k, jnp.bool_(False)
    )

    ngrp = (batch * dict_size) // _G
    acts2d = acts.reshape(ngrp, _G)
    gmaxf = gmax.transpose(1, 0, 2).reshape(ngrp)

    sc_round = pl.kernel(
        _sc_select_body,
        out_type=jax.ShapeDtypeStruct((_NW, _NB), jnp.int32),
        mesh=plsc.VectorSubcoreMesh(core_axis_name="c", subcore_axis_name="s"),
        compiler_params=pltpu.CompilerParams(
            needs_layout_passes=False, use_tc_tiling_on_sc=False),
        scratch_types=[
            pltpu.VMEM((16,), jnp.int32),
            pltpu.VMEM((ngrp // _NW,), jnp.float32),
            pltpu.VMEM((ngrp // _NW,), jnp.int32),
            pltpu.VMEM((_CH, _G), jnp.float32),
            pltpu.VMEM((16 * _NB,), jnp.int32),
            pltpu.VMEM((_NB,), jnp.int32),
            pltpu.SemaphoreType.DMA,
        ],
    )

    for _ in range(3):
        w = hi - lo
        s = jnp.maximum(0, (32 - jax.lax.clz(w - 1)) - 10)
        params = jnp.zeros((16,), jnp.int32).at[0].set(lo).at[1].set(s)
        hist = sc_round(params, gmaxf, acts2d)
        lo, hi, dead = _sc_update(lo, hi, s, hist, k, dead)

    vk_bits = jnp.where(dead, jnp.int32(0), hi)

    nkt = dict_size // _BK
    topk, sae, stats = pl.pallas_call(
        functools.partial(_decode_kernel, nbt=nbt, nkt=nkt),
        grid=(nbt, nkt),
        in_specs=[
            pl.BlockSpec(memory_space=pltpu.SMEM),
            pl.BlockSpec((_BM, _BK), lambda i, t: (i, t)),
            pl.BlockSpec((_BK, _ACT), lambda i, t: (t, 0)),
            pl.BlockSpec((1, _ACT), lambda i, t: (0, 0)),
            pl.BlockSpec((_BM, _ACT), lambda i, t: (i, 0)),
        ],
        out_specs=[
            pl.BlockSpec((_BM, _BK), lambda i, t: (i, t)),
            pl.BlockSpec((_BM, _ACT), lambda i, t: (i, 0)),
            pl.BlockSpec(memory_space=pltpu.SMEM),
        ],
        out_shape=[
            jax.ShapeDtypeStruct((batch, dict_size), jnp.float32),
            jax.ShapeDtypeStruct((batch, _ACT), jnp.float32),
            jax.ShapeDtypeStruct((8,), jnp.float32),
        ],
        scratch_shapes=[pltpu.VMEM((_BM, _ACT), jnp.float32)],
    )(vk_bits.reshape(1), acts, W_dec, b2, x)

    l1_norm = stats[0] / batch
    l0_norm = stats[1] / batch
    minpos = stats[2]
    l2_loss = stats[3] / (batch * _ACT)
    l1_loss = jnp.float32(_L1_COEFF) * l1_norm
    loss = l2_loss + l1_loss
    threshold = jnp.where(jnp.isfinite(minpos), minpos, jnp.float32(0.0))

    return (sae, topk, loss, l2_loss, l1_loss, l0_norm, l1_norm, threshold)


# speedup vs baseline: 1.2229x; 1.0857x over previous
import functools

import jax
import jax.numpy as jnp
from jax.experimental import pallas as pl
from jax.experimental.pallas import tpu as pltpu
from jax.experimental.pallas import tpu_sc as plsc

_ACT = 768
_TOPK = 64
_L1_COEFF = 0.0008
_INF_BITS = 0x7F800000

_BM = 512
_BN = 1536
_BK = 1536

_LADDER = (0, 0x3F000000, 0x3F628F5C, 0x3F6CCCCD, 0x3FC00000, 0x41000000)
_D = len(_LADDER)

_G = 16
_NW = 32
_NB = 1024
_CH = 128


def _encode_kernel(x_ref, w_ref, b_ref, sel_ref, acts_ref, gmax_ref,
                   cnt_ref):
    i = pl.program_id(0)
    j = pl.program_id(1)
    xc = x_ref[...] - b_ref[...]
    a = jnp.maximum(
        jnp.dot(xc, w_ref[...], preferred_element_type=jnp.float32), 0.0
    )
    acts_ref[...] = a

    m = a
    for sh in (8, 4, 2, 1):
        m = jnp.maximum(m, pltpu.roll(m, sh, axis=1))
    gmax_ref[...] = jnp.dot(m, sel_ref[...],
                            preferred_element_type=jnp.float32)[None]

    @pl.when(jnp.logical_and(i == 0, j == 0))
    def _init():
        for t in range(_D):
            cnt_ref[t] = 0

    bits = jax.lax.bitcast_convert_type(a, jnp.int32)
    for t in range(_D):
        cnt_ref[t] += jnp.sum(bits > _LADDER[t], dtype=jnp.int32)


def _sc_select_body(params_hbm, gmax_hbm, acts_hbm, hist_hbm,
                    params_v, gmaxs_v, idx_v, rows_v, hist_v, outb_v, sem):
    gpw = gmax_hbm.shape[0] // _NW
    cid = jax.lax.axis_index("c")
    sid = jax.lax.axis_index("s")
    wid = sid * 2 + cid
    lane = jax.lax.iota(jnp.int32, 16)

    pltpu.sync_copy(params_hbm, params_v)
    pv = params_v[...]
    zero = jnp.int32(0)
    lo = jnp.sum(jnp.where(lane == 0, pv, zero))
    s = jnp.sum(jnp.where(lane == 1, pv, zero))

    pltpu.sync_copy(gmax_hbm.at[pl.ds(wid * gpw, gpw)], gmaxs_v)

    def zh(i, c):
        hist_v[pl.ds(i * 16, 16)] = jnp.zeros((16,), jnp.int32)
        return c

    jax.lax.fori_loop(0, (16 * _NB) // 16, zh, 0, unroll=8)

    def scan(i, off_v):
        g = plsc.bitcast(gmaxs_v[pl.ds(i * 16, 16)], jnp.int32)
        m = g > lo
        inc = jnp.where(m, jnp.int32(1), jnp.int32(0))
        pos = off_v + plsc.cumsum(inc) - 1
        ids = (wid * gpw + i * 16) + lane
        plsc.store_scatter(idx_v, [pos], ids, mask=m)
        return off_v + plsc.all_reduce_population_count(m)

    off_v = jax.lax.fori_loop(
        0, gpw // 16, scan, jnp.zeros((16,), jnp.int32), unroll=4
    )
    n = jnp.sum(jnp.where(lane == 0, off_v, zero))

    padv = jnp.full((16,), wid * gpw, jnp.int32)
    for pj in range(_CH // 16):
        posp = n + pj * 16 + lane
        plsc.store_scatter(idx_v, [posp], padv, mask=posp < gpw)

    ntrip = (n + _CH - 1) // _CH
    ones = jnp.ones((16,), jnp.int32)
    lanenb = lane * _NB

    def chunk(c2, carry):
        pltpu.async_copy(
            acts_hbm.at[idx_v.at[pl.ds(c2 * _CH, _CH)]], rows_v, sem
        ).wait()
        nvalid = n - c2 * _CH

        def hrow(j, c3):
            v = plsc.bitcast(rows_v[j, :], jnp.int32)
            m = jnp.logical_and(v > lo, j < nvalid)
            rel = v - (lo + 1)
            b = jnp.minimum(jax.lax.shift_right_logical(rel, s), _NB - 1)
            plsc.addupdate_scatter(hist_v, [b + lanenb], ones, mask=m)
            return c3

        jax.lax.fori_loop(0, _CH, hrow, 0, unroll=8)
        return carry

    jax.lax.fori_loop(0, ntrip, chunk, 0)

    def merge(q, c):
        acc = jnp.zeros((16,), jnp.int32)
        for l in range(16):
            acc = acc + hist_v[pl.ds(l * _NB + q * 16, 16)]
        outb_v[pl.ds(q * 16, 16)] = acc
        return c

    jax.lax.fori_loop(0, _NB // 16, merge, 0)
    pltpu.sync_copy(outb_v, hist_hbm.at[wid])


def _decode_kernel(vk_ref, acts_ref, w_ref, b_ref, x_ref,
                   topk_ref, sae_ref, stat_ref, acc_ref, nbt, nkt):
    i = pl.program_id(0)
    kt = pl.program_id(1)
    a = acts_ref[...]
    bits = jax.lax.bitcast_convert_type(a, jnp.int32)
    m = jnp.where(bits >= vk_ref[0], a, 0.0)
    topk_ref[...] = m

    @pl.when(kt == 0)
    def _zero_acc():
        acc_ref[...] = jnp.zeros_like(acc_ref)

    acc_ref[...] += jnp.dot(m, w_ref[...], preferred_element_type=jnp.float32)

    @pl.when(jnp.logical_and(i == 0, kt == 0))
    def _init_stats():
        stat_ref[0] = 0.0
        stat_ref[1] = 0.0
        stat_ref[2] = jnp.inf
        stat_ref[3] = 0.0

    stat_ref[0] += jnp.sum(m)
    stat_ref[1] += jnp.sum(m > 0, dtype=jnp.float32)
    stat_ref[2] = jnp.minimum(
        stat_ref[2], jnp.min(jnp.where(m > 0, m, jnp.inf))
    )

    @pl.when(kt == nkt - 1)
    def _finish_row():
        sae = acc_ref[...] + b_ref[...]
        sae_ref[...] = sae
        d = sae - x_ref[...]
        stat_ref[3] += jnp.sum(d * d)


def _bracket_update(thrs, hi, counts, k, dead):
    ge = counts >= k
    idx = jnp.arange(_D, dtype=jnp.int32)
    jstar = jnp.max(jnp.where(ge, idx, -1))
    dead = jnp.logical_or(dead, jstar < 0)
    js = jnp.maximum(jstar, 0)
    new_lo = thrs[js]
    new_hi = jnp.where(js < _D - 1, thrs[jnp.minimum(js + 1, _D - 1)], hi)
    return new_lo, new_hi, dead


def _sc_update(lo, hi, s, hist, k, dead):
    tot = jnp.sum(hist, axis=0)
    suf = jnp.cumsum(tot[::-1])[::-1]
    ge = suf >= k
    idx = jnp.arange(_NB, dtype=jnp.int32)
    b = jnp.max(jnp.where(ge, idx, -1))
    dead = jnp.logical_or(dead, b < 0)
    bs = jnp.maximum(b, 0)
    w = hi - lo
    new_lo = lo + (bs << s)
    new_hi = lo + jnp.minimum((bs + 1) << s, w)
    return new_lo, new_hi, dead


def kernel(x, W_enc, W_dec, b_dec):
    batch = x.shape[0]
    dict_size = W_enc.shape[1]
    k = _TOPK * batch
    b2 = b_dec.reshape(1, _ACT)

    nbt = batch // _BM
    ndt = dict_size // _BN
    ng = _BN // _G
    cidx = jnp.arange(ng)
    sel = jnp.zeros((_BN, ng), jnp.float32).at[cidx * _G + _G - 1, cidx].set(1.0)

    acts, gmax, cnt0 = pl.pallas_call(
        _encode_kernel,
        grid=(nbt, ndt),
        in_specs=[
            pl.BlockSpec((_BM, _ACT), lambda i, j: (i, 0)),
            pl.BlockSpec((_ACT, _BN), lambda i, j: (0, j)),
            pl.BlockSpec((1, _ACT), lambda i, j: (0, 0)),
            pl.BlockSpec((_BN, ng), lambda i, j: (0, 0)),
        ],
        out_specs=[
            pl.BlockSpec((_BM, _BN), lambda i, j: (i, j)),
            pl.BlockSpec((1, _BM, ng), lambda i, j: (j, i, 0)),
            pl.BlockSpec(memory_space=pltpu.SMEM),
        ],
        out_shape=[
            jax.ShapeDtypeStruct((batch, dict_size), jnp.float32),
            jax.ShapeDtypeStruct((ndt, batch, ng), jnp.float32),
            jax.ShapeDtypeStruct((_D,), jnp.int32),
        ],
    )(x, W_enc, b2, sel)

    thrs0 = jnp.array(_LADDER, dtype=jnp.int32)
    lo, hi, dead = _bracket_update(
        thrs0, jnp.int32(_INF_BITS), cnt0, k, jnp.bool_(False)
    )

    ngrp = (batch * dict_size) // _G
    acts2d = acts.reshape(ngrp, _G)
    gmaxf = gmax.transpose(1, 0, 2).reshape(ngrp)

    sc_round = pl.kernel(
        _sc_select_body,
        out_type=jax.ShapeDtypeStruct((_NW, _NB), jnp.int32),
        mesh=plsc.VectorSubcoreMesh(core_axis_name="c", subcore_axis_name="s"),
        compiler_params=pltpu.CompilerParams(
            needs_layout_passes=False, use_tc_tiling_on_sc=False),
        scratch_types=[
            pltpu.VMEM((16,), jnp.int32),
            pltpu.VMEM((ngrp // _NW,), jnp.float32),
            pltpu.VMEM((ngrp // _NW,), jnp.int32),
            pltpu.VMEM((_CH, _G), jnp.float32),
            pltpu.VMEM((16 * _NB,), jnp.int32),
            pltpu.VMEM((_NB,), jnp.int32),
            pltpu.SemaphoreType.DMA,
        ],
    )

    for _ in range(3):
        w = hi - lo
        s = jnp.maximum(0, (32 - jax.lax.clz(w - 1)) - 10)
        params = jnp.zeros((16,), jnp.int32).at[0].set(lo).at[1].set(s)
        hist = sc_round(params, gmaxf, acts2d)
        lo, hi, dead = _sc_update(lo, hi, s, hist, k, dead)

    vk_bits = jnp.where(dead, jnp.int32(0), hi)

    nkt = dict_size // _BK
    topk, sae, stats = pl.pallas_call(
        functools.partial(_decode_kernel, nbt=nbt, nkt=nkt),
        grid=(nbt, nkt),
        in_specs=[
            pl.BlockSpec(memory_space=pltpu.SMEM),
            pl.BlockSpec((_BM, _BK), lambda i, t: (i, t)),
            pl.BlockSpec((_BK, _ACT), lambda i, t: (t, 0)),
            pl.BlockSpec((1, _ACT), lambda i, t: (0, 0)),
            pl.BlockSpec((_BM, _ACT), lambda i, t: (i, 0)),
        ],
        out_specs=[
            pl.BlockSpec((_BM, _BK), lambda i, t: (i, t)),
            pl.BlockSpec((_BM, _ACT), lambda i, t: (i, 0)),
            pl.BlockSpec(memory_space=pltpu.SMEM),
        ],
        out_shape=[
            jax.ShapeDtypeStruct((batch, dict_size), jnp.float32),
            jax.ShapeDtypeStruct((batch, _ACT), jnp.float32),
            jax.ShapeDtypeStruct((8,), jnp.float32),
        ],
        scratch_shapes=[pltpu.VMEM((_BM, _ACT), jnp.float32)],
    )(vk_bits.reshape(1), acts, W_dec, b2, x)

    l1_norm = stats[0] / batch
    l0_norm = stats[1] / batch
    minpos = stats[2]
    l2_loss = stats[3] / (batch * _ACT)
    l1_loss = jnp.float32(_L1_COEFF) * l1_norm
    loss = l2_loss + l1_loss
    threshold = jnp.where(jnp.isfinite(minpos), minpos, jnp.float32(0.0))

    return (sae, topk, loss, l2_loss, l1_loss, l0_norm, l1_norm, threshold)
